# baseline (speedup 1.0000x reference)
import functools

import jax
import jax.numpy as jnp
from jax import lax
from jax.experimental import pallas as pl
from jax.experimental.pallas import tpu as pltpu
from jax.experimental.pallas import tpu_sc as plsc

N = 10000
NPAD = 10240
E = 160000
CK = 64
ECHUNKS = 2560
EPAD = ECHUNKS * CK
NS = 16
NC = 2
ZR = NPAD // NS
CR = NPAD // NS

_mesh = plsc.VectorSubcoreMesh(
    core_axis_name="c", subcore_axis_name="s", num_cores=NC, num_subcores=NS)


def _wid():
    return lax.axis_index("c"), lax.axis_index("s")


@functools.partial(
    pl.kernel,
    out_type=jax.ShapeDtypeStruct((NC, NPAD, 128), jnp.float32),
    mesh=_mesh,
    scratch_types=[
        pltpu.VMEM((EPAD // NC // NS // CK, CK), jnp.int32),
        pltpu.VMEM((CK, 128), jnp.float32),
        pltpu.VMEM_SHARED((NPAD, 128), jnp.float32),
        pltpu.SemaphoreType.DMA,
    ],
)
def _k1_deg(dst_hbm, ones_hbm, zeros_hbm, deg_hbm, dstb, onesb, acc, ssem):
    c, t = _wid()
    nch = EPAD // NC // NS // CK
    pltpu.sync_copy(dst_hbm.at[pl.ds(c * (nch * NS) + t * nch, nch)], dstb)
    pltpu.sync_copy(ones_hbm, onesb)
    pltpu.sync_copy(zeros_hbm, acc.at[pl.ds(t * ZR, ZR)])
    plsc.subcore_barrier()

    @pl.loop(0, nch)
    def _(j):
        pltpu.async_copy(onesb, acc.at[dstb.at[j]], ssem, add=True)

    @pl.loop(0, nch)
    def _(j):
        pltpu.make_async_copy(onesb, acc.at[dstb.at[j]], ssem).wait()

    plsc.subcore_barrier()
    pltpu.sync_copy(acc.at[pl.ds(t * CR, CR)],
                    deg_hbm.at[c].at[pl.ds(t * CR, CR)])


def _k2_body(x_ref, w_ref, da_ref, db_ref, gp_ref, dis_ref):
    h = jnp.dot(x_ref[...], w_ref[...], preferred_element_type=jnp.float32)
    deg = 1.0 + da_ref[:, 0:1] + db_ref[:, 0:1]
    dis = lax.rsqrt(deg)
    g = h * dis
    for k in range(4):
        gp_ref[k] = g[:, k * 128:(k + 1) * 128]
    dis_ref[...] = dis


def _k2(x, wcat, dega, degb):
    blk = 1000
    return pl.pallas_call(
        _k2_body,
        grid=(N // blk,),
        in_specs=[pl.BlockSpec((blk, 256), lambda i: (i, 0)),
                  pl.BlockSpec((256, 512), lambda i: (0, 0)),
                  pl.BlockSpec((blk, 128), lambda i: (i, 0)),
                  pl.BlockSpec((blk, 128), lambda i: (i, 0))],
        out_specs=[pl.BlockSpec((4, blk, 128), lambda i: (0, i, 0)),
                   pl.BlockSpec((blk, 1), lambda i: (i, 0))],
        out_shape=[jax.ShapeDtypeStruct((4, NPAD, 128), jnp.float32),
                   jax.ShapeDtypeStruct((N, 1), jnp.float32)],
    )(x, wcat, dega, degb)


@functools.partial(
    pl.kernel,
    out_type=jax.ShapeDtypeStruct((4, NPAD, 128), jnp.float32),
    mesh=_mesh,
    scratch_types=[
        pltpu.VMEM((ECHUNKS // NS // 4, CK), jnp.int32),
        pltpu.VMEM((ECHUNKS // NS // 4, CK), jnp.int32),
        pltpu.VMEM((4, CK, 128), jnp.float32),
        pltpu.VMEM_SHARED((NPAD, 128), jnp.float32),
        pltpu.SemaphoreType.DMA,
        pltpu.SemaphoreType.DMA,
        pltpu.SemaphoreType.DMA,
        pltpu.SemaphoreType.DMA,
        pltpu.SemaphoreType.DMA,
        pltpu.SemaphoreType.DMA,
        pltpu.SemaphoreType.DMA,
        pltpu.SemaphoreType.DMA,
    ],
)
def _k3_agg(gp_hbm, src_hbm, dst_hbm, zeros_hbm, s_hbm,
            srcb, dstb, rowsb, acc, gs0, gs1, gs2, gs3, ss0, ss1, ss2, ss3):
    gsem = (gs0, gs1, gs2, gs3)
    ssem = (ss0, ss1, ss2, ss3)
    NB = 4
    c, t = _wid()
    nch = ECHUNKS // NS // 4
    for p in range(2):
        fp = c * 2 + p
        pltpu.sync_copy(zeros_hbm, acc.at[pl.ds(t * ZR, ZR)])
        plsc.subcore_barrier()
        for seg in range(4):
            base = t * (4 * nch) + seg * nch
            pltpu.sync_copy(src_hbm.at[pl.ds(base, nch)], srcb)
            pltpu.sync_copy(dst_hbm.at[pl.ds(base, nch)], dstb)
            for b in range(NB):
                pltpu.async_copy(gp_hbm.at[fp].at[srcb.at[b]], rowsb.at[b],
                                 gsem[b])

            @pl.loop(0, nch // NB)
            def _(g):
                for b in range(NB):
                    j = g * NB + b
                    pltpu.make_async_copy(gp_hbm.at[fp].at[srcb.at[j]],
                                          rowsb.at[b], gsem[b]).wait()
                    pltpu.async_copy(rowsb.at[b], acc.at[dstb.at[j]],
                                     ssem[b], add=True)
                for b in range(NB):
                    j = g * NB + b
                    pltpu.make_async_copy(rowsb.at[b], acc.at[dstb.at[j]],
                                          ssem[b]).wait()
                    nj = j + NB

                    @pl.when(nj < nch)
                    def _():
                        pltpu.async_copy(gp_hbm.at[fp].at[srcb.at[nj]],
                                         rowsb.at[b], gsem[b])

        plsc.subcore_barrier()
        pltpu.sync_copy(acc.at[pl.ds(t * CR, CR)],
                        s_hbm.at[fp].at[pl.ds(t * CR, CR)])
        plsc.subcore_barrier()


def _k4_body(gp_ref, sp_ref, dis_ref, w2_ref, bc_ref, g2_ref):
    dis = dis_ref[...]
    parts = []
    for k in range(4):
        parts.append(jnp.maximum(
            (gp_ref[k] + sp_ref[k]) * dis + bc_ref[:, k * 128:(k + 1) * 128],
            0.0))
    pcat = jnp.concatenate(parts, axis=1)
    z = jnp.dot(pcat, w2_ref[...], preferred_element_type=jnp.float32)
    g2_ref[...] = z * dis


def _k4(gp, sp, dis, w2cat, bcat):
    blk = 1000
    return pl.pallas_call(
        _k4_body,
        grid=(N // blk,),
        in_specs=[pl.BlockSpec((4, blk, 128), lambda i: (0, i, 0)),
                  pl.BlockSpec((4, blk, 128), lambda i: (0, i, 0)),
                  pl.BlockSpec((blk, 1), lambda i: (i, 0)),
                  pl.BlockSpec((512, 128), lambda i: (0, 0)),
                  pl.BlockSpec((1, 512), lambda i: (0, 0))],
        out_specs=pl.BlockSpec((blk, 128), lambda i: (i, 0)),
        out_shape=jax.ShapeDtypeStruct((N, 128), jnp.float32),
    )(gp, sp, dis, w2cat, bcat)


@functools.partial(
    pl.kernel,
    out_type=jax.ShapeDtypeStruct((NC, NPAD, 128), jnp.float32),
    mesh=_mesh,
    scratch_types=[
        pltpu.VMEM((EPAD // NC // NS // CK // 2, CK), jnp.int32),
        pltpu.VMEM((EPAD // NC // NS // CK // 2, CK), jnp.int32),
        pltpu.VMEM((4, CK, 128), jnp.float32),
        pltpu.VMEM_SHARED((NPAD, 128), jnp.float32),
        pltpu.SemaphoreType.DMA,
        pltpu.SemaphoreType.DMA,
        pltpu.SemaphoreType.DMA,
        pltpu.SemaphoreType.DMA,
        pltpu.SemaphoreType.DMA,
        pltpu.SemaphoreType.DMA,
        pltpu.SemaphoreType.DMA,
        pltpu.SemaphoreType.DMA,
    ],
)
def _k5_agg2(g2_hbm, src_hbm, dst_hbm, zeros_hbm, s2_hbm,
             srcb, dstb, rowsb, acc,
             gs0, gs1, gs2, gs3, ss0, ss1, ss2, ss3):
    gsem = (gs0, gs1, gs2, gs3)
    ssem = (ss0, ss1, ss2, ss3)
    NB = 4
    c, t = _wid()
    nch = EPAD // NC // NS // CK // 2
    pltpu.sync_copy(zeros_hbm, acc.at[pl.ds(t * ZR, ZR)])
    plsc.subcore_barrier()
    for seg in range(2):
        base = c * (2 * nch * NS) + t * (2 * nch) + seg * nch
        pltpu.sync_copy(src_hbm.at[pl.ds(base, nch)], srcb)
        pltpu.sync_copy(dst_hbm.at[pl.ds(base, nch)], dstb)
        for b in range(NB):
            pltpu.async_copy(g2_hbm.at[srcb.at[b]], rowsb.at[b], gsem[b])

        @pl.loop(0, nch // NB)
        def _(g):
            for b in range(NB):
                j = g * NB + b
                pltpu.make_async_copy(g2_hbm.at[srcb.at[j]], rowsb.at[b],
                                      gsem[b]).wait()
                pltpu.async_copy(rowsb.at[b], acc.at[dstb.at[j]], ssem[b],
                                 add=True)
            for b in range(NB):
                j = g * NB + b
                pltpu.make_async_copy(rowsb.at[b], acc.at[dstb.at[j]],
                                      ssem[b]).wait()
                nj = j + NB

                @pl.when(nj < nch)
                def _():
                    pltpu.async_copy(g2_hbm.at[srcb.at[nj]], rowsb.at[b],
                                     gsem[b])

    plsc.subcore_barrier()
    pltpu.sync_copy(acc.at[pl.ds(t * CR, CR)],
                    s2_hbm.at[c].at[pl.ds(t * CR, CR)])


def _k6_body(s2a_ref, s2b_ref, g2_ref, dis_ref, b2_ref, out_ref):
    out_ref[...] = ((s2a_ref[...] + s2b_ref[...] + g2_ref[...])
                    * dis_ref[...] + b2_ref[...])


def _k6(s2a, s2b, g2, dis, b2row):
    blk = 1000
    return pl.pallas_call(
        _k6_body,
        grid=(N // blk,),
        in_specs=[pl.BlockSpec((blk, 128), lambda i: (i, 0)),
                  pl.BlockSpec((blk, 128), lambda i: (i, 0)),
                  pl.BlockSpec((blk, 128), lambda i: (i, 0)),
                  pl.BlockSpec((blk, 1), lambda i: (i, 0)),
                  pl.BlockSpec((1, 128), lambda i: (0, 0))],
        out_specs=pl.BlockSpec((blk, 128), lambda i: (i, 0)),
        out_shape=jax.ShapeDtypeStruct((N, 128), jnp.float32),
    )(s2a, s2b, g2, dis, b2row)


def kernel(x, edge_index, W_B1, b_B1, W_B2, b_B2, W_Y1, b_Y1, W_Y2, b_Y2):
    src = edge_index[0].astype(jnp.int32)
    dst = edge_index[1].astype(jnp.int32)
    npad = EPAD - E
    srcp = jnp.concatenate([src, jnp.zeros((npad,), jnp.int32)]
                           ).reshape(ECHUNKS, CK)
    pad_dst = N + (jnp.arange(npad, dtype=jnp.int32) % (NPAD - N))
    dstp = jnp.concatenate([dst, pad_dst]).reshape(ECHUNKS, CK)

    wcat = jnp.concatenate([W_B1, W_Y1], axis=1)
    bcat = jnp.concatenate([b_B1, b_Y1]).reshape(1, 512)
    z256 = jnp.zeros((256, 1), jnp.float32)
    w2cat = jnp.concatenate(
        [jnp.concatenate([W_B2, z256]),
         jnp.concatenate([z256, W_Y2]),
         jnp.zeros((512, 126), jnp.float32)], axis=1)
    b2row = jnp.concatenate([b_B2, b_Y2, jnp.zeros((126,), jnp.float32)]
                            ).reshape(1, 128)

    ones128 = jnp.ones((CK, 128), jnp.float32)
    zeros128 = jnp.zeros((ZR, 128), jnp.float32)

    degp = _k1_deg(dstp, ones128, zeros128)
    gp, dis = _k2(x, wcat, degp[0], degp[1])
    sp = _k3_agg(gp, srcp, dstp, zeros128)
    g2 = _k4(gp, sp, dis, w2cat, bcat)
    s2p = _k5_agg2(g2, srcp, dstp, zeros128)
    out = _k6(s2p[0], s2p[1], g2, dis, b2row)
    return (out[:, 0:1], out[:, 1:2])


# speedup vs baseline: 1.0289x; 1.0289x over previous
import functools

import jax
import jax.numpy as jnp
from jax import lax
from jax.experimental import pallas as pl
from jax.experimental.pallas import tpu as pltpu
from jax.experimental.pallas import tpu_sc as plsc

N = 10000
NPAD = 10240
E = 160000
CK = 64
ECHUNKS = 2560
EPAD = ECHUNKS * CK
NS = 16
NC = 2
ZR = NPAD // NS
CR = NPAD // NS

_mesh = plsc.VectorSubcoreMesh(
    core_axis_name="c", subcore_axis_name="s", num_cores=NC, num_subcores=NS)


def _wid():
    return lax.axis_index("c"), lax.axis_index("s")


@functools.partial(
    pl.kernel,
    out_type=jax.ShapeDtypeStruct((NC, NPAD, 128), jnp.float32),
    mesh=_mesh,
    scratch_types=[
        pltpu.VMEM((EPAD // NC // NS // CK, CK), jnp.int32),
        pltpu.VMEM((CK, 128), jnp.float32),
        pltpu.VMEM_SHARED((NPAD, 128), jnp.float32),
        pltpu.SemaphoreType.DMA,
    ],
)
def _k1_deg(dst_hbm, ones_hbm, zeros_hbm, deg_hbm, dstb, onesb, acc, ssem):
    c, t = _wid()
    nch = EPAD // NC // NS // CK
    pltpu.sync_copy(dst_hbm.at[pl.ds(c * (nch * NS) + t * nch, nch)], dstb)
    pltpu.sync_copy(ones_hbm, onesb)
    pltpu.sync_copy(zeros_hbm, acc.at[pl.ds(t * ZR, ZR)])
    plsc.subcore_barrier()

    @pl.loop(0, nch)
    def _(j):
        pltpu.async_copy(onesb, acc.at[dstb.at[j]], ssem, add=True)

    @pl.loop(0, nch)
    def _(j):
        pltpu.make_async_copy(onesb, acc.at[dstb.at[j]], ssem).wait()

    plsc.subcore_barrier()
    pltpu.sync_copy(acc.at[pl.ds(t * CR, CR)],
                    deg_hbm.at[c].at[pl.ds(t * CR, CR)])


def _k2_body(x_ref, w_ref, da_ref, db_ref, gp_ref, dis_ref):
    h = jnp.dot(x_ref[...], w_ref[...], preferred_element_type=jnp.float32)
    deg = 1.0 + da_ref[:, 0:1] + db_ref[:, 0:1]
    dis = lax.rsqrt(deg)
    g = h * dis
    for k in range(4):
        gp_ref[k] = g[:, k * 128:(k + 1) * 128]
    dis_ref[...] = dis


def _k2(x, wcat, dega, degb):
    blk = 1000
    return pl.pallas_call(
        _k2_body,
        grid=(N // blk,),
        in_specs=[pl.BlockSpec((blk, 256), lambda i: (i, 0)),
                  pl.BlockSpec((256, 512), lambda i: (0, 0)),
                  pl.BlockSpec((blk, 128), lambda i: (i, 0)),
                  pl.BlockSpec((blk, 128), lambda i: (i, 0))],
        out_specs=[pl.BlockSpec((4, blk, 128), lambda i: (0, i, 0)),
                   pl.BlockSpec((blk, 1), lambda i: (i, 0))],
        out_shape=[jax.ShapeDtypeStruct((4, NPAD, 128), jnp.float32),
                   jax.ShapeDtypeStruct((N, 1), jnp.float32)],
    )(x, wcat, dega, degb)


@functools.partial(
    pl.kernel,
    out_type=jax.ShapeDtypeStruct((4, NPAD, 128), jnp.float32),
    mesh=_mesh,
    scratch_types=[
        pltpu.VMEM((ECHUNKS // NS // 4, CK), jnp.int32),
        pltpu.VMEM((ECHUNKS // NS // 4, CK), jnp.int32),
        pltpu.VMEM((4, CK, 128), jnp.float32),
        pltpu.VMEM_SHARED((NPAD, 128), jnp.float32),
        pltpu.SemaphoreType.DMA,
        pltpu.SemaphoreType.DMA,
        pltpu.SemaphoreType.DMA,
        pltpu.SemaphoreType.DMA,
        pltpu.SemaphoreType.DMA,
        pltpu.SemaphoreType.DMA,
        pltpu.SemaphoreType.DMA,
        pltpu.SemaphoreType.DMA,
    ],
)
def _k3_agg(gp_hbm, src_hbm, dst_hbm, zeros_hbm, s_hbm,
            srcb, dstb, rowsb, acc, gs0, gs1, gs2, gs3, ss0, ss1, ss2, ss3):
    gsem = (gs0, gs1, gs2, gs3)
    ssem = (ss0, ss1, ss2, ss3)
    NB = 4
    c, t = _wid()
    nch = ECHUNKS // NS // 4
    for p in range(2):
        fp = c * 2 + p
        pltpu.sync_copy(zeros_hbm, acc.at[pl.ds(t * ZR, ZR)])
        plsc.subcore_barrier()
        for seg in range(4):
            base = t * (4 * nch) + seg * nch
            pltpu.sync_copy(src_hbm.at[pl.ds(base, nch)], srcb)
            pltpu.sync_copy(dst_hbm.at[pl.ds(base, nch)], dstb)
            for b in range(NB):
                pltpu.async_copy(gp_hbm.at[fp].at[srcb.at[b]], rowsb.at[b],
                                 gsem[b])

            @pl.loop(0, nch // NB)
            def _(g):
                for b in range(NB):
                    j = g * NB + b
                    pltpu.make_async_copy(gp_hbm.at[fp].at[srcb.at[j]],
                                          rowsb.at[b], gsem[b]).wait()
                    pltpu.async_copy(rowsb.at[b], acc.at[dstb.at[j]],
                                     ssem[b], add=True)
                for b in range(NB):
                    j = g * NB + b
                    pltpu.make_async_copy(rowsb.at[b], acc.at[dstb.at[j]],
                                          ssem[b]).wait()
                    nj = j + NB

                    @pl.when(nj < nch)
                    def _():
                        pltpu.async_copy(gp_hbm.at[fp].at[srcb.at[nj]],
                                         rowsb.at[b], gsem[b])

        plsc.subcore_barrier()
        pltpu.sync_copy(acc.at[pl.ds(t * CR, CR)],
                        s_hbm.at[fp].at[pl.ds(t * CR, CR)])
        plsc.subcore_barrier()


def _k4_body(gp_ref, sp_ref, dis_ref, w2_ref, bc_ref, g2_ref):
    dis = dis_ref[...]
    parts = []
    for k in range(4):
        parts.append(jnp.maximum(
            (gp_ref[k] + sp_ref[k]) * dis + bc_ref[:, k * 128:(k + 1) * 128],
            0.0))
    pcat = jnp.concatenate(parts, axis=1)
    z = jnp.dot(pcat, w2_ref[...], preferred_element_type=jnp.float32)
    g2_ref[0] = z * dis
    g2_ref[1] = z * dis


def _k4(gp, sp, dis, w2cat, bcat):
    blk = 1000
    return pl.pallas_call(
        _k4_body,
        grid=(N // blk,),
        in_specs=[pl.BlockSpec((4, blk, 128), lambda i: (0, i, 0)),
                  pl.BlockSpec((4, blk, 128), lambda i: (0, i, 0)),
                  pl.BlockSpec((blk, 1), lambda i: (i, 0)),
                  pl.BlockSpec((512, 128), lambda i: (0, 0)),
                  pl.BlockSpec((1, 512), lambda i: (0, 0))],
        out_specs=pl.BlockSpec((2, blk, 128), lambda i: (0, i, 0)),
        out_shape=jax.ShapeDtypeStruct((2, N, 128), jnp.float32),
    )(gp, sp, dis, w2cat, bcat)


@functools.partial(
    pl.kernel,
    out_type=jax.ShapeDtypeStruct((NC, NPAD, 128), jnp.float32),
    mesh=_mesh,
    scratch_types=[
        pltpu.VMEM((EPAD // NC // NS // CK // 2, CK), jnp.int32),
        pltpu.VMEM((EPAD // NC // NS // CK // 2, CK), jnp.int32),
        pltpu.VMEM((4, CK, 128), jnp.float32),
        pltpu.VMEM_SHARED((NPAD, 128), jnp.float32),
        pltpu.SemaphoreType.DMA,
        pltpu.SemaphoreType.DMA,
        pltpu.SemaphoreType.DMA,
        pltpu.SemaphoreType.DMA,
        pltpu.SemaphoreType.DMA,
        pltpu.SemaphoreType.DMA,
        pltpu.SemaphoreType.DMA,
        pltpu.SemaphoreType.DMA,
    ],
)
def _k5_agg2(g2_hbm, src_hbm, dst_hbm, zeros_hbm, s2_hbm,
             srcb, dstb, rowsb, acc,
             gs0, gs1, gs2, gs3, ss0, ss1, ss2, ss3):
    gsem = (gs0, gs1, gs2, gs3)
    ssem = (ss0, ss1, ss2, ss3)
    NB = 4
    c, t = _wid()
    nch = EPAD // NC // NS // CK // 2
    pltpu.sync_copy(zeros_hbm, acc.at[pl.ds(t * ZR, ZR)])
    plsc.subcore_barrier()
    for seg in range(2):
        base = c * (2 * nch * NS) + t * (2 * nch) + seg * nch
        pltpu.sync_copy(src_hbm.at[pl.ds(base, nch)], srcb)
        pltpu.sync_copy(dst_hbm.at[pl.ds(base, nch)], dstb)
        for b in range(NB):
            pltpu.async_copy(g2_hbm.at[c].at[srcb.at[b]], rowsb.at[b], gsem[b])

        @pl.loop(0, nch // NB)
        def _(g):
            for b in range(NB):
                j = g * NB + b
                pltpu.make_async_copy(g2_hbm.at[c].at[srcb.at[j]],
                                      rowsb.at[b], gsem[b]).wait()
                pltpu.async_copy(rowsb.at[b], acc.at[dstb.at[j]], ssem[b],
                                 add=True)
            for b in range(NB):
                j = g * NB + b
                pltpu.make_async_copy(rowsb.at[b], acc.at[dstb.at[j]],
                                      ssem[b]).wait()
                nj = j + NB

                @pl.when(nj < nch)
                def _():
                    pltpu.async_copy(g2_hbm.at[c].at[srcb.at[nj]],
                                     rowsb.at[b], gsem[b])

    plsc.subcore_barrier()
    pltpu.sync_copy(acc.at[pl.ds(t * CR, CR)],
                    s2_hbm.at[c].at[pl.ds(t * CR, CR)])


def _k6_body(s2a_ref, s2b_ref, g2_ref, dis_ref, b2_ref, out_ref):
    out_ref[...] = ((s2a_ref[...] + s2b_ref[...] + g2_ref[...])
                    * dis_ref[...] + b2_ref[...])


def _k6(s2a, s2b, g2, dis, b2row):
    blk = 1000
    return pl.pallas_call(
        _k6_body,
        grid=(N // blk,),
        in_specs=[pl.BlockSpec((blk, 128), lambda i: (i, 0)),
                  pl.BlockSpec((blk, 128), lambda i: (i, 0)),
                  pl.BlockSpec((blk, 128), lambda i: (i, 0)),
                  pl.BlockSpec((blk, 1), lambda i: (i, 0)),
                  pl.BlockSpec((1, 128), lambda i: (0, 0))],
        out_specs=pl.BlockSpec((blk, 128), lambda i: (i, 0)),
        out_shape=jax.ShapeDtypeStruct((N, 128), jnp.float32),
    )(s2a, s2b, g2, dis, b2row)


def kernel(x, edge_index, W_B1, b_B1, W_B2, b_B2, W_Y1, b_Y1, W_Y2, b_Y2):
    src = edge_index[0].astype(jnp.int32)
    dst = edge_index[1].astype(jnp.int32)
    npad = EPAD - E
    srcp = jnp.concatenate([src, jnp.zeros((npad,), jnp.int32)]
                           ).reshape(ECHUNKS, CK)
    pad_dst = N + (jnp.arange(npad, dtype=jnp.int32) % (NPAD - N))
    dstp = jnp.concatenate([dst, pad_dst]).reshape(ECHUNKS, CK)

    wcat = jnp.concatenate([W_B1, W_Y1], axis=1)
    bcat = jnp.concatenate([b_B1, b_Y1]).reshape(1, 512)
    z256 = jnp.zeros((256, 1), jnp.float32)
    w2cat = jnp.concatenate(
        [jnp.concatenate([W_B2, z256]),
         jnp.concatenate([z256, W_Y2]),
         jnp.zeros((512, 126), jnp.float32)], axis=1)
    b2row = jnp.concatenate([b_B2, b_Y2, jnp.zeros((126,), jnp.float32)]
                            ).reshape(1, 128)

    ones128 = jnp.ones((CK, 128), jnp.float32)
    zeros128 = jnp.zeros((ZR, 128), jnp.float32)

    degp = _k1_deg(dstp, ones128, zeros128)
    gp, dis = _k2(x, wcat, degp[0], degp[1])
    sp = _k3_agg(gp, srcp, dstp, zeros128)
    g2 = _k4(gp, sp, dis, w2cat, bcat)
    s2p = _k5_agg2(g2, srcp, dstp, zeros128)
    out = _k6(s2p[0], s2p[1], g2[0], dis, b2row)
    return (out[:, 0:1], out[:, 1:2])


# speedup vs baseline: 1.1283x; 1.0967x over previous
import functools

import jax
import jax.numpy as jnp
from jax import lax
from jax.experimental import pallas as pl
from jax.experimental.pallas import tpu as pltpu
from jax.experimental.pallas import tpu_sc as plsc

N = 10000
NPAD = 10240
E = 160000
CK = 64
ECHUNKS = 2560
EPAD = ECHUNKS * CK
NS = 16
NC = 2
ZR = NPAD // NS
CR = NPAD // NS

_mesh = plsc.VectorSubcoreMesh(
    core_axis_name="c", subcore_axis_name="s", num_cores=NC, num_subcores=NS)


def _wid():
    return lax.axis_index("c"), lax.axis_index("s")


@functools.partial(
    pl.kernel,
    out_type=jax.ShapeDtypeStruct((NC, NPAD, 128), jnp.float32),
    mesh=_mesh,
    scratch_types=[
        pltpu.VMEM((EPAD // NC // NS // CK, CK), jnp.int32),
        pltpu.VMEM((CK, 128), jnp.float32),
        pltpu.VMEM_SHARED((NPAD, 128), jnp.float32),
        pltpu.SemaphoreType.DMA,
    ],
)
def _k1_deg(dst_hbm, ones_hbm, zeros_hbm, deg_hbm, dstb, onesb, acc, ssem):
    c, t = _wid()
    nch = EPAD // NC // NS // CK
    pltpu.sync_copy(dst_hbm.at[pl.ds(c * (nch * NS) + t * nch, nch)], dstb)
    pltpu.sync_copy(ones_hbm, onesb)
    pltpu.sync_copy(zeros_hbm, acc.at[pl.ds(t * ZR, ZR)])
    plsc.subcore_barrier()

    @pl.loop(0, nch)
    def _(j):
        pltpu.async_copy(onesb, acc.at[dstb.at[j]], ssem, add=True)

    @pl.loop(0, nch)
    def _(j):
        pltpu.make_async_copy(onesb, acc.at[dstb.at[j]], ssem).wait()

    plsc.subcore_barrier()
    pltpu.sync_copy(acc.at[pl.ds(t * CR, CR)],
                    deg_hbm.at[c].at[pl.ds(t * CR, CR)])


def _k2_body(x_ref, w_ref, da_ref, db_ref, gp_ref, dis_ref):
    h = jnp.dot(x_ref[...], w_ref[...], preferred_element_type=jnp.float32)
    deg = 1.0 + da_ref[0, :, 0:1] + db_ref[0, :, 0:1]
    dis = lax.rsqrt(deg)
    g = h * dis
    for k in range(4):
        gp_ref[k] = g[:, k * 128:(k + 1) * 128]
    dis_ref[...] = dis


def _k2(x, wcat, dega, degb):
    blk = 1000
    return pl.pallas_call(
        _k2_body,
        grid=(N // blk,),
        in_specs=[pl.BlockSpec((blk, 256), lambda i: (i, 0)),
                  pl.BlockSpec((256, 512), lambda i: (0, 0)),
                  pl.BlockSpec((1, blk, 128), lambda i: (0, i, 0)),
                  pl.BlockSpec((1, blk, 128), lambda i: (1, i, 0))],
        out_specs=[pl.BlockSpec((4, blk, 128), lambda i: (0, i, 0)),
                   pl.BlockSpec((blk, 1), lambda i: (i, 0))],
        out_shape=[jax.ShapeDtypeStruct((4, NPAD, 128), jnp.float32),
                   jax.ShapeDtypeStruct((N, 1), jnp.float32)],
    )(x, wcat, dega, degb)


@functools.partial(
    pl.kernel,
    out_type=jax.ShapeDtypeStruct((4, NPAD, 128), jnp.float32),
    mesh=_mesh,
    scratch_types=[
        pltpu.VMEM((ECHUNKS // NS // 4, CK), jnp.int32),
        pltpu.VMEM((ECHUNKS // NS // 4, CK), jnp.int32),
        pltpu.VMEM((4, CK, 128), jnp.float32),
        pltpu.VMEM_SHARED((NPAD, 128), jnp.float32),
        pltpu.SemaphoreType.DMA,
        pltpu.SemaphoreType.DMA,
        pltpu.SemaphoreType.DMA,
        pltpu.SemaphoreType.DMA,
        pltpu.SemaphoreType.DMA,
        pltpu.SemaphoreType.DMA,
        pltpu.SemaphoreType.DMA,
        pltpu.SemaphoreType.DMA,
    ],
)
def _k3_agg(gp_hbm, src_hbm, dst_hbm, zeros_hbm, s_hbm,
            srcb, dstb, rowsb, acc, gs0, gs1, gs2, gs3, ss0, ss1, ss2, ss3):
    gsem = (gs0, gs1, gs2, gs3)
    ssem = (ss0, ss1, ss2, ss3)
    NB = 4
    c, t = _wid()
    nch = ECHUNKS // NS // 4
    for p in range(2):
        fp = c * 2 + p
        pltpu.sync_copy(zeros_hbm, acc.at[pl.ds(t * ZR, ZR)])
        plsc.subcore_barrier()
        for seg in range(4):
            base = t * (4 * nch) + seg * nch
            pltpu.sync_copy(src_hbm.at[pl.ds(base, nch)], srcb)
            pltpu.sync_copy(dst_hbm.at[pl.ds(base, nch)], dstb)
            for b in range(NB):
                pltpu.async_copy(gp_hbm.at[fp].at[srcb.at[b]], rowsb.at[b],
                                 gsem[b])

            @pl.loop(0, nch // NB)
            def _(g):
                for b in range(NB):
                    j = g * NB + b
                    pltpu.make_async_copy(gp_hbm.at[fp].at[srcb.at[j]],
                                          rowsb.at[b], gsem[b]).wait()
                    pltpu.async_copy(rowsb.at[b], acc.at[dstb.at[j]],
                                     ssem[b], add=True)
                for b in range(NB):
                    j = g * NB + b
                    pltpu.make_async_copy(rowsb.at[b], acc.at[dstb.at[j]],
                                          ssem[b]).wait()
                    nj = j + NB

                    @pl.when(nj < nch)
                    def _():
                        pltpu.async_copy(gp_hbm.at[fp].at[srcb.at[nj]],
                                         rowsb.at[b], gsem[b])

        plsc.subcore_barrier()
        pltpu.sync_copy(acc.at[pl.ds(t * CR, CR)],
                        s_hbm.at[fp].at[pl.ds(t * CR, CR)])
        plsc.subcore_barrier()


def _k4_body(gp_ref, sp_ref, dis_ref, w2_ref, bc_ref, g2_ref):
    dis = dis_ref[...]
    parts = []
    for k in range(4):
        parts.append(jnp.maximum(
            (gp_ref[k] + sp_ref[k]) * dis + bc_ref[:, k * 128:(k + 1) * 128],
            0.0))
    pcat = jnp.concatenate(parts, axis=1)
    z = jnp.dot(pcat, w2_ref[...], preferred_element_type=jnp.float32)
    g2_ref[0] = z * dis
    g2_ref[1] = z * dis


def _k4(gp, sp, dis, w2cat, bcat):
    blk = 1000
    return pl.pallas_call(
        _k4_body,
        grid=(N // blk,),
        in_specs=[pl.BlockSpec((4, blk, 128), lambda i: (0, i, 0)),
                  pl.BlockSpec((4, blk, 128), lambda i: (0, i, 0)),
                  pl.BlockSpec((blk, 1), lambda i: (i, 0)),
                  pl.BlockSpec((512, 128), lambda i: (0, 0)),
                  pl.BlockSpec((1, 512), lambda i: (0, 0))],
        out_specs=pl.BlockSpec((2, blk, 128), lambda i: (0, i, 0)),
        out_shape=jax.ShapeDtypeStruct((2, N, 128), jnp.float32),
    )(gp, sp, dis, w2cat, bcat)


@functools.partial(
    pl.kernel,
    out_type=jax.ShapeDtypeStruct((NC, NPAD, 128), jnp.float32),
    mesh=_mesh,
    scratch_types=[
        pltpu.VMEM((EPAD // NC // NS // CK // 2, CK), jnp.int32),
        pltpu.VMEM((EPAD // NC // NS // CK // 2, CK), jnp.int32),
        pltpu.VMEM((4, CK, 128), jnp.float32),
        pltpu.VMEM_SHARED((NPAD, 128), jnp.float32),
        pltpu.SemaphoreType.DMA,
        pltpu.SemaphoreType.DMA,
        pltpu.SemaphoreType.DMA,
        pltpu.SemaphoreType.DMA,
        pltpu.SemaphoreType.DMA,
        pltpu.SemaphoreType.DMA,
        pltpu.SemaphoreType.DMA,
        pltpu.SemaphoreType.DMA,
    ],
)
def _k5_agg2(g2_hbm, src_hbm, dst_hbm, zeros_hbm, s2_hbm,
             srcb, dstb, rowsb, acc,
             gs0, gs1, gs2, gs3, ss0, ss1, ss2, ss3):
    gsem = (gs0, gs1, gs2, gs3)
    ssem = (ss0, ss1, ss2, ss3)
    NB = 4
    c, t = _wid()
    nch = EPAD // NC // NS // CK // 2
    pltpu.sync_copy(zeros_hbm, acc.at[pl.ds(t * ZR, ZR)])
    plsc.subcore_barrier()
    for seg in range(2):
        base = c * (2 * nch * NS) + t * (2 * nch) + seg * nch
        pltpu.sync_copy(src_hbm.at[pl.ds(base, nch)], srcb)
        pltpu.sync_copy(dst_hbm.at[pl.ds(base, nch)], dstb)
        for b in range(NB):
            pltpu.async_copy(g2_hbm.at[c].at[srcb.at[b]], rowsb.at[b], gsem[b])

        @pl.loop(0, nch // NB)
        def _(g):
            for b in range(NB):
                j = g * NB + b
                pltpu.make_async_copy(g2_hbm.at[c].at[srcb.at[j]],
                                      rowsb.at[b], gsem[b]).wait()
                pltpu.async_copy(rowsb.at[b], acc.at[dstb.at[j]], ssem[b],
                                 add=True)
            for b in range(NB):
                j = g * NB + b
                pltpu.make_async_copy(rowsb.at[b], acc.at[dstb.at[j]],
                                      ssem[b]).wait()
                nj = j + NB

                @pl.when(nj < nch)
                def _():
                    pltpu.async_copy(g2_hbm.at[c].at[srcb.at[nj]],
                                     rowsb.at[b], gsem[b])

    plsc.subcore_barrier()
    pltpu.sync_copy(acc.at[pl.ds(t * CR, CR)],
                    s2_hbm.at[c].at[pl.ds(t * CR, CR)])


def _k6_body(s2a_ref, s2b_ref, g2_ref, dis_ref, b2_ref, ob_ref, oy_ref):
    v = ((s2a_ref[0] + s2b_ref[0] + g2_ref[0])
         * dis_ref[...] + b2_ref[...])
    ob_ref[...] = v[:, 0:1]
    oy_ref[...] = v[:, 1:2]


def _k6(s2p, g2, dis, b2row):
    blk = 1000
    return pl.pallas_call(
        _k6_body,
        grid=(N // blk,),
        in_specs=[pl.BlockSpec((1, blk, 128), lambda i: (0, i, 0)),
                  pl.BlockSpec((1, blk, 128), lambda i: (1, i, 0)),
                  pl.BlockSpec((1, blk, 128), lambda i: (0, i, 0)),
                  pl.BlockSpec((blk, 1), lambda i: (i, 0)),
                  pl.BlockSpec((1, 128), lambda i: (0, 0))],
        out_specs=[pl.BlockSpec((blk, 1), lambda i: (i, 0)),
                   pl.BlockSpec((blk, 1), lambda i: (i, 0))],
        out_shape=[jax.ShapeDtypeStruct((N, 1), jnp.float32),
                   jax.ShapeDtypeStruct((N, 1), jnp.float32)],
    )(s2p, s2p, g2, dis, b2row)


def kernel(x, edge_index, W_B1, b_B1, W_B2, b_B2, W_Y1, b_Y1, W_Y2, b_Y2):
    src = edge_index[0].astype(jnp.int32)
    dst = edge_index[1].astype(jnp.int32)
    npad = EPAD - E
    srcp = jnp.concatenate([src, jnp.zeros((npad,), jnp.int32)]
                           ).reshape(ECHUNKS, CK)
    pad_dst = N + (jnp.arange(npad, dtype=jnp.int32) % (NPAD - N))
    dstp = jnp.concatenate([dst, pad_dst]).reshape(ECHUNKS, CK)

    wcat = jnp.concatenate([W_B1, W_Y1], axis=1)
    bcat = jnp.concatenate([b_B1, b_Y1]).reshape(1, 512)
    z256 = jnp.zeros((256, 1), jnp.float32)
    w2cat = jnp.concatenate(
        [jnp.concatenate([W_B2, z256]),
         jnp.concatenate([z256, W_Y2]),
         jnp.zeros((512, 126), jnp.float32)], axis=1)
    b2row = jnp.concatenate([b_B2, b_Y2, jnp.zeros((126,), jnp.float32)]
                            ).reshape(1, 128)

    ones128 = jnp.ones((CK, 128), jnp.float32)
    zeros128 = jnp.zeros((ZR, 128), jnp.float32)

    degp = _k1_deg(dstp, ones128, zeros128)
    gp, dis = _k2(x, wcat, degp, degp)
    sp = _k3_agg(gp, srcp, dstp, zeros128)
    g2 = _k4(gp, sp, dis, w2cat, bcat)
    s2p = _k5_agg2(g2, srcp, dstp, zeros128)
    ob, oy = _k6(s2p, g2, dis, b2row)
    return (ob, oy)
